# Initial kernel scaffold; baseline (speedup 1.0000x reference)
import jax
import jax.numpy as jnp
from jax.experimental import pallas as pl


def kernel(centers, positives, negatives, W_in, W_out):
    raise NotImplementedError("write your pallas kernel here")


# speedup vs baseline: 5.2880x; 5.2880x over previous
import functools

import jax
import jax.numpy as jnp
from jax import lax
from jax.experimental import pallas as pl
from jax.experimental.pallas import tpu as pltpu
from jax.experimental.pallas import tpu_sc as plsc

B = 16384
D = 64
K = 20
NC = 2
NS = 16
NW = NC * NS
RPW = B // NW
NPW = RPW * K
CH = 16
NEGC = CH * K
NCHUNK = RPW // CH
LS = D // 16


def _dot16(a_ref, a_row, b_ref, b_row):
    acc = a_ref[a_row, pl.ds(0, 16)] * b_ref[b_row, pl.ds(0, 16)]
    for c in range(1, LS):
        acc = acc + a_ref[a_row, pl.ds(c * 16, 16)] * b_ref[b_row, pl.ds(c * 16, 16)]
    return jnp.sum(acc)


def _sc_body(centers_hbm, positives_hbm, negs_hbm, win_hbm, wout_hbm,
             pos_hbm, negsc_hbm,
             cidx, pidx, nidx, cbuf, pbuf, nbuf, pos_o, neg_o, sem):
    wid = lax.axis_index("s") * NC + lax.axis_index("c")
    base = wid * RPW
    nbase = wid * NPW

    pltpu.sync_copy(centers_hbm.at[pl.ds(base, RPW)], cidx)
    pltpu.sync_copy(positives_hbm.at[pl.ds(base, RPW)], pidx)
    pltpu.sync_copy(negs_hbm.at[pl.ds(nbase, NPW)], nidx)

    cps = [
        pltpu.async_copy(win_hbm.at[cidx.at[pl.ds(c * 128, 128)]],
                         cbuf.at[pl.ds(c * 128, 128)], sem)
        for c in range(RPW // 128)
    ]
    for cp in cps:
        cp.wait()

    def chunk(j, _):
        ro = j * CH
        no = j * NEGC
        cp_p = pltpu.async_copy(wout_hbm.at[pidx.at[pl.ds(ro, CH)]], pbuf, sem)
        cps_n = [
            pltpu.async_copy(wout_hbm.at[nidx.at[pl.ds(no + o, sz)]],
                             nbuf.at[pl.ds(o, sz)], sem)
            for o, sz in ((0, 128), (128, 128), (256, 64))
        ]
        cp_p.wait()
        for cp in cps_n:
            cp.wait()

        lanes = lax.iota(jnp.int32, 16)

        def row_body(r, pos_vec):
            row = ro + r
            pos_vec = jnp.where(lanes == r, _dot16(cbuf, row, pbuf, r), pos_vec)
            v0 = jnp.zeros((16,), jnp.float32)
            v1 = jnp.zeros((16,), jnp.float32)
            for k in range(16):
                v0 = jnp.where(lanes == k, _dot16(cbuf, row, nbuf, r * K + k), v0)
            for k in range(16, K):
                v1 = jnp.where(lanes == k - 16,
                               _dot16(cbuf, row, nbuf, r * K + k), v1)
            nb = no + r * K
            plsc.store_scatter(neg_o, [nb + lanes], v0)
            plsc.store_scatter(neg_o, [nb + 16 + lanes], v1, mask=lanes < K - 16)
            return pos_vec

        pos_vec = lax.fori_loop(0, CH, row_body, jnp.zeros((16,), jnp.float32))
        pos_o[pl.ds(ro, CH)] = pos_vec
        return 0

    lax.fori_loop(0, NCHUNK, chunk, 0)

    pltpu.sync_copy(pos_o, pos_hbm.at[pl.ds(base, RPW)])
    pltpu.sync_copy(neg_o, negsc_hbm.at[pl.ds(nbase, NPW)])


_sc_scores = pl.kernel(
    _sc_body,
    out_type=(
        jax.ShapeDtypeStruct((B,), jnp.float32),
        jax.ShapeDtypeStruct((B * K,), jnp.float32),
    ),
    mesh=plsc.VectorSubcoreMesh(
        core_axis_name="c", subcore_axis_name="s",
        num_cores=NC, num_subcores=NS,
    ),
    compiler_params=pltpu.CompilerParams(
        needs_layout_passes=False, use_tc_tiling_on_sc=False,
    ),
    scratch_types=[
        pltpu.VMEM((RPW,), jnp.int32),
        pltpu.VMEM((RPW,), jnp.int32),
        pltpu.VMEM((NPW,), jnp.int32),
        pltpu.VMEM((RPW, D), jnp.float32),
        pltpu.VMEM((CH, D), jnp.float32),
        pltpu.VMEM((NEGC, D), jnp.float32),
        pltpu.VMEM((RPW,), jnp.float32),
        pltpu.VMEM((NPW,), jnp.float32),
        pltpu.SemaphoreType.DMA,
    ],
)


def _log_sigmoid(x):
    return jnp.minimum(x, 0.0) - jnp.log1p(jnp.exp(-jnp.abs(x)))


def _loss_body(pos_ref, neg_ref, out_ref):
    ls_pos = jnp.sum(_log_sigmoid(pos_ref[...]))
    ls_neg = jnp.sum(_log_sigmoid(-neg_ref[...]))
    out_ref[0, 0] = -(ls_pos + ls_neg) / B


_loss = pl.pallas_call(
    _loss_body,
    out_shape=jax.ShapeDtypeStruct((1, 1), jnp.float32),
    out_specs=pl.BlockSpec(memory_space=pltpu.SMEM),
)


@jax.jit
def kernel(centers, positives, negatives, W_in, W_out):
    pos_s, neg_s = _sc_scores(centers, positives, negatives.reshape(-1),
                              W_in, W_out)
    total = _loss(pos_s.reshape(128, B // 128), neg_s.reshape(B * K // 128, 128))
    return total[0, 0]
